# Initial kernel scaffold; baseline (speedup 1.0000x reference)
import jax
import jax.numpy as jnp
from jax.experimental import pallas as pl


def kernel(day_ids, time_ids, location_ids, timedelta_ids, day_table, time_table, location_table, timedelta_table, ln_scale, ln_bias):
    raise NotImplementedError("write your pallas kernel here")


# speedup vs baseline: 1.0459x; 1.0459x over previous
import functools

import jax
import jax.numpy as jnp
from jax import lax
from jax.experimental import pallas as pl
from jax.experimental.pallas import tpu as pltpu
from jax.experimental.pallas import tpu_sc as plsc

EMBED = 128
LANES = 16
CHUNK = 128


def _rsqrt(x):
    xi = plsc.bitcast(x, jnp.int32)
    yi = jnp.int32(0x5F3759DF) - lax.shift_right_logical(xi, 1)
    y = plsc.bitcast(yi, jnp.float32)
    half = x * jnp.float32(-0.5)
    for _ in range(3):
        y = y * (jnp.float32(1.5) + half * y * y)
    return y


def _sc_body(n_tokens, day_ids, time_ids, loc_ids, td_ids,
             day_t, time_t, loc_t, td_t, scale, bias, out,
             day_tab, time_tab, td_tab, scale_v, bias_v,
             day_i, time_i, td_i, loc_i, loc_rows, colbuf, out_buf, sem):
    info = plsc.get_sparse_core_info()
    nw = info.num_cores * info.num_subcores
    wid = lax.axis_index("s") * info.num_cores + lax.axis_index("c")
    per_tile = n_tokens // nw
    base = wid * per_tile

    pltpu.sync_copy(day_t, day_tab)
    pltpu.sync_copy(time_t, time_tab)
    pltpu.sync_copy(td_t, td_tab)
    pltpu.sync_copy(scale, scale_v)
    pltpu.sync_copy(bias, bias_v)

    iota = lax.iota(jnp.int32, LANES)
    inv_d = jnp.float32(1.0 / EMBED)
    eps = jnp.float32(1e-6)

    def chunk_body(c, _):
        off = base + c * CHUNK
        pltpu.sync_copy(day_ids.at[pl.ds(off, CHUNK)], day_i)
        pltpu.sync_copy(time_ids.at[pl.ds(off, CHUNK)], time_i)
        pltpu.sync_copy(td_ids.at[pl.ds(off, CHUNK)], td_i)
        pltpu.sync_copy(loc_ids.at[pl.ds(off, CHUNK)], loc_i)
        pltpu.async_copy(loc_t.at[loc_i], loc_rows, sem).wait()

        def group_body(g, _):
            tok0 = g * LANES
            row_i = tok0 + iota
            row_off = lax.shift_left(row_i, 7)
            day_off = lax.shift_left(day_i[pl.ds(tok0, LANES)], 7)
            time_off = lax.shift_left(time_i[pl.ds(tok0, LANES)], 7)
            td_off = lax.shift_left(td_i[pl.ds(tok0, LANES)], 7)

            def col1(d, carry):
                s, q = carry
                dsp = jnp.full((LANES,), d, jnp.int32)
                a = plsc.load_gather(day_tab, [day_off + d])
                a = a + plsc.load_gather(time_tab, [time_off + d])
                a = a + plsc.load_gather(td_tab, [td_off + d])
                a = a + plsc.load_gather(loc_rows, [row_i, dsp])
                colbuf[pl.ds(d * LANES, LANES)] = a
                return s + a, q + a * a


            zero = jnp.zeros((LANES,), jnp.float32)
            s, q = lax.fori_loop(0, EMBED, col1, (zero, zero))
            mean = s * inv_d
            var = q * inv_d - mean * mean
            inv = _rsqrt(var + eps)

            def col2(d, _):
                dsp = jnp.full((LANES,), d, jnp.int32)
                x = colbuf[pl.ds(d * LANES, LANES)]
                gam = plsc.load_gather(scale_v, [dsp])
                bet = plsc.load_gather(bias_v, [dsp])
                y = (x - mean) * inv * gam + bet
                plsc.store_scatter(out_buf, [row_off + d], y)
                return 0

            lax.fori_loop(0, EMBED, col2, 0)
            return 0

        lax.fori_loop(0, CHUNK // LANES, group_body, 0)
        pltpu.sync_copy(out_buf, out.at[pl.ds(off * EMBED, CHUNK * EMBED)])
        return 0

    lax.fori_loop(0, per_tile // CHUNK, chunk_body, 0)


def kernel(day_ids, time_ids, location_ids, timedelta_ids,
           day_table, time_table, location_table, timedelta_table,
           ln_scale, ln_bias):
    b, l = day_ids.shape
    n = b * l
    flat = lambda x: x.reshape(n).astype(jnp.int32)

    mesh = plsc.VectorSubcoreMesh(core_axis_name="c", subcore_axis_name="s")
    run = pl.kernel(
        functools.partial(_sc_body, n),
        out_type=jax.ShapeDtypeStruct((n * EMBED,), jnp.float32),
        mesh=mesh,
        scratch_types=[
            pltpu.VMEM((day_table.size,), jnp.float32),
            pltpu.VMEM((time_table.size,), jnp.float32),
            pltpu.VMEM((timedelta_table.size,), jnp.float32),
            pltpu.VMEM((EMBED,), jnp.float32),
            pltpu.VMEM((EMBED,), jnp.float32),
            pltpu.VMEM((CHUNK,), jnp.int32),
            pltpu.VMEM((CHUNK,), jnp.int32),
            pltpu.VMEM((CHUNK,), jnp.int32),
            pltpu.VMEM((CHUNK,), jnp.int32),
            pltpu.VMEM((CHUNK, EMBED), jnp.float32),
            pltpu.VMEM((EMBED * LANES,), jnp.float32),
            pltpu.VMEM((CHUNK * EMBED,), jnp.float32),
            pltpu.SemaphoreType.DMA,
        ],
        compiler_params=pltpu.CompilerParams(needs_layout_passes=False),
    )
    out = run(flat(day_ids), flat(time_ids), flat(location_ids),
              flat(timedelta_ids),
              day_table.reshape(-1), time_table.reshape(-1),
              location_table, timedelta_table.reshape(-1),
              ln_scale, ln_bias)
    return out.reshape(b, l, EMBED)


# speedup vs baseline: 1.0609x; 1.0143x over previous
import functools

import jax
import jax.numpy as jnp
from jax import lax
from jax.experimental import pallas as pl
from jax.experimental.pallas import tpu as pltpu
from jax.experimental.pallas import tpu_sc as plsc

EMBED = 128
LANES = 16
CHUNK = 128
UNROLL = 16


def _rsqrt(x):
    xi = plsc.bitcast(x, jnp.int32)
    yi = jnp.int32(0x5F3759DF) - lax.shift_right_logical(xi, 1)
    y = plsc.bitcast(yi, jnp.float32)
    half = x * jnp.float32(-0.5)
    for _ in range(3):
        y = y * (jnp.float32(1.5) + half * y * y)
    return y


def _sc_body(n_tokens, day_ids, time_ids, loc_ids, td_ids,
             day_t, time_t, loc_t, td_t, scale, bias, out,
             day_tab, time_tab, td_tab, scale_v, bias_v,
             day_i, time_i, td_i, loc_i, loc_rows, colbuf, out_buf, sem):
    info = plsc.get_sparse_core_info()
    nw = info.num_cores * info.num_subcores
    wid = lax.axis_index("s") * info.num_cores + lax.axis_index("c")
    per_tile = n_tokens // nw
    base = wid * per_tile

    pltpu.sync_copy(day_t, day_tab)
    pltpu.sync_copy(time_t, time_tab)
    pltpu.sync_copy(td_t, td_tab)
    pltpu.sync_copy(scale, scale_v)
    pltpu.sync_copy(bias, bias_v)

    iota = lax.iota(jnp.int32, LANES)
    inv_d = jnp.float32(1.0 / EMBED)
    eps = jnp.float32(1e-6)

    def chunk_body(c, _):
        off = base + c * CHUNK
        pltpu.sync_copy(day_ids.at[pl.ds(off, CHUNK)], day_i)
        pltpu.sync_copy(time_ids.at[pl.ds(off, CHUNK)], time_i)
        pltpu.sync_copy(td_ids.at[pl.ds(off, CHUNK)], td_i)
        pltpu.sync_copy(loc_ids.at[pl.ds(off, CHUNK)], loc_i)
        pltpu.async_copy(loc_t.at[loc_i], loc_rows, sem).wait()

        def group_body(g, _):
            tok0 = g * LANES
            row_i = tok0 + iota
            row_off = lax.shift_left(row_i, 7)
            day_off = lax.shift_left(day_i[pl.ds(tok0, LANES)], 7)
            time_off = lax.shift_left(time_i[pl.ds(tok0, LANES)], 7)
            td_off = lax.shift_left(td_i[pl.ds(tok0, LANES)], 7)

            def col1(blk, carry):
                s, q = carry
                d0 = blk * UNROLL
                dsp0 = jnp.full((LANES,), d0, jnp.int32)
                for j in range(UNROLL):
                    d = d0 + j
                    a = plsc.load_gather(day_tab, [day_off + d])
                    a = a + plsc.load_gather(time_tab, [time_off + d])
                    a = a + plsc.load_gather(td_tab, [td_off + d])
                    a = a + plsc.load_gather(loc_rows, [row_i, dsp0 + j])
                    colbuf[pl.ds(d0 * LANES + j * LANES, LANES)] = a
                    s = s + a
                    q = q + a * a
                return s, q

            zero = jnp.zeros((LANES,), jnp.float32)
            s, q = lax.fori_loop(0, EMBED // UNROLL, col1, (zero, zero))
            mean = s * inv_d
            var = q * inv_d - mean * mean
            inv = _rsqrt(var + eps)

            def col2(blk, _):
                d0 = blk * UNROLL
                dsp0 = jnp.full((LANES,), d0, jnp.int32)
                for j in range(UNROLL):
                    d = d0 + j
                    x = colbuf[pl.ds(d0 * LANES + j * LANES, LANES)]
                    gam = plsc.load_gather(scale_v, [dsp0 + j])
                    bet = plsc.load_gather(bias_v, [dsp0 + j])
                    y = (x - mean) * inv * gam + bet
                    plsc.store_scatter(out_buf, [row_off + d], y)
                return 0

            lax.fori_loop(0, EMBED // UNROLL, col2, 0)
            return 0

        lax.fori_loop(0, CHUNK // LANES, group_body, 0)
        pltpu.sync_copy(out_buf, out.at[pl.ds(off * EMBED, CHUNK * EMBED)])
        return 0

    lax.fori_loop(0, per_tile // CHUNK, chunk_body, 0)


def kernel(day_ids, time_ids, location_ids, timedelta_ids,
           day_table, time_table, location_table, timedelta_table,
           ln_scale, ln_bias):
    b, l = day_ids.shape
    n = b * l
    flat = lambda x: x.reshape(n).astype(jnp.int32)

    mesh = plsc.VectorSubcoreMesh(core_axis_name="c", subcore_axis_name="s")
    run = pl.kernel(
        functools.partial(_sc_body, n),
        out_type=jax.ShapeDtypeStruct((n * EMBED,), jnp.float32),
        mesh=mesh,
        scratch_types=[
            pltpu.VMEM((day_table.size,), jnp.float32),
            pltpu.VMEM((time_table.size,), jnp.float32),
            pltpu.VMEM((timedelta_table.size,), jnp.float32),
            pltpu.VMEM((EMBED,), jnp.float32),
            pltpu.VMEM((EMBED,), jnp.float32),
            pltpu.VMEM((CHUNK,), jnp.int32),
            pltpu.VMEM((CHUNK,), jnp.int32),
            pltpu.VMEM((CHUNK,), jnp.int32),
            pltpu.VMEM((CHUNK,), jnp.int32),
            pltpu.VMEM((CHUNK, EMBED), jnp.float32),
            pltpu.VMEM((EMBED * LANES,), jnp.float32),
            pltpu.VMEM((CHUNK * EMBED,), jnp.float32),
            pltpu.SemaphoreType.DMA,
        ],
        compiler_params=pltpu.CompilerParams(needs_layout_passes=False),
    )
    out = run(flat(day_ids), flat(time_ids), flat(location_ids),
              flat(timedelta_ids),
              day_table.reshape(-1), time_table.reshape(-1),
              location_table, timedelta_table.reshape(-1),
              ln_scale, ln_bias)
    return out.reshape(b, l, EMBED)
